# baseline (speedup 1.0000x reference)
import jax
import jax.numpy as jnp
from jax.experimental import pallas as pl
from jax.experimental.pallas import tpu as pltpu


def _reduce_body(xyz_ref, pts_ref, ox_ref, op_ref):
    j = pl.program_id(1)
    mx = jnp.max(xyz_ref[0], axis=-1)[None, None, :]
    mp = jnp.max(pts_ref[0], axis=-1)[None, None, :]

    @pl.when(j == 0)
    def _():
        ox_ref[...] = mx
        op_ref[...] = mp

    @pl.when(j > 0)
    def _():
        ox_ref[...] = jnp.maximum(ox_ref[...], mx)
        op_ref[...] = jnp.maximum(op_ref[...], mp)


def kernel(xyz, points):
    B, C, N = xyz.shape
    D = points.shape[1]
    NCHUNK = 4
    CH = N // NCHUNK

    ox, op = pl.pallas_call(
        _reduce_body,
        grid=(B, NCHUNK),
        in_specs=[
            pl.BlockSpec((1, C, CH), lambda b, j: (b, 0, j)),
            pl.BlockSpec((1, D, CH), lambda b, j: (b, 0, j)),
        ],
        out_specs=[
            pl.BlockSpec((1, 1, C), lambda b, j: (b, 0, 0)),
            pl.BlockSpec((1, 1, D), lambda b, j: (b, 0, 0)),
        ],
        out_shape=[
            jax.ShapeDtypeStruct((B, 1, C), xyz.dtype),
            jax.ShapeDtypeStruct((B, 1, D), points.dtype),
        ],
        compiler_params=pltpu.CompilerParams(
            dimension_semantics=("parallel", "arbitrary"),
        ),
    )(xyz, points)

    new_points = jnp.concatenate([ox, op], axis=-1).reshape(B, 1, C + D)
    new_points = jnp.transpose(new_points, (0, 2, 1))
    new_xyz = jnp.zeros((B, C, 1), dtype=xyz.dtype)
    return (new_xyz, new_points)
